# baseline (speedup 1.0000x reference)
import dataclasses
import functools

import numpy as np

import jax
import jax.numpy as jnp
from jax import lax
from jax.experimental import pallas as pl
from jax.experimental.pallas import tpu as pltpu
from jax.experimental.pallas import tpu_sc as plsc

N_NODES = 10000
N_EDGES = 320000
DIM = 128

NC = 2
NS = 16
NW = NC * NS

C = 80
E_PAD = 327680
CHUNKS = E_PAD // C
CPT = CHUNKS // NW
NPH = 4
PC = CPT // NPH
NBUF = 4
NOBUF = 2
LOOK = 3
AGG_ROWS = 10240
ROWS_PT = AGG_ROWS // NS

_PERM = np.array([p * 64 + g * 16 + i
                  for g in range(DIM // 32)
                  for p in range(2)
                  for i in range(16)], dtype=np.int32)


def _sc_aggregate(hb, src2d, dst2d, w2d):
    mesh = plsc.VectorSubcoreMesh(core_axis_name="c", subcore_axis_name="s")

    def bcast_lane(vec, i):
        dnums = lax.GatherDimensionNumbers(
            offset_dims=(), collapsed_slice_dims=(0,), start_index_map=(0,))
        idx = jnp.full((16, 1), i, jnp.int32)
        return lax.gather(vec, idx, dnums, (1,),
                          mode=lax.GatherScatterMode.PROMISE_IN_BOUNDS)

    cp = pltpu.CompilerParams()
    if "needs_layout_passes" in pltpu.CompilerParams.__dataclass_fields__:
        cp = dataclasses.replace(cp, needs_layout_passes=False)
    cp = dataclasses.replace(cp, use_tc_tiling_on_sc=False)

    @functools.partial(
        pl.kernel,
        out_type=jax.ShapeDtypeStruct((NC, AGG_ROWS, DIM), jnp.float32),
        mesh=mesh,
        compiler_params=cp,
        scratch_types=[
            pltpu.VMEM((PC, C), jnp.int32),
            pltpu.VMEM((PC, C), jnp.int32),
            pltpu.VMEM((PC, C), jnp.float32),
            pltpu.VMEM((NBUF, C, DIM // 2), jnp.int32),
            pltpu.VMEM((NOBUF, C, DIM), jnp.float32),
            pltpu.VMEM_SHARED((AGG_ROWS, DIM), jnp.float32),
            pltpu.SemaphoreType.DMA((NBUF,)),
            pltpu.SemaphoreType.DMA((NOBUF,)),
        ],
    )
    def body(h_hbm, src_hbm, dst_hbm, w_hbm, out_hbm,
             src_v, dst_v, w_v, rows_v, obuf, agg_sh, gsem, ssem):
        cid = lax.axis_index("c")
        sid = lax.axis_index("s")
        wid = sid * NC + cid
        base = wid * CPT

        def gather(jj, b):
            pltpu.async_copy(h_hbm.at[src_v.at[jj]], rows_v.at[b], gsem.at[b])

        def gather_wait(b):
            pltpu.make_async_copy(
                h_hbm.at[src_v.at[0]], rows_v.at[b], gsem.at[b]).wait()

        def scatter(jj, ob):
            pltpu.async_copy(obuf.at[ob], agg_sh.at[dst_v.at[jj]],
                             ssem.at[ob], add=True)

        def scatter_wait(ob):
            pltpu.make_async_copy(
                obuf.at[ob], agg_sh.at[dst_v.at[0]], ssem.at[ob]).wait()

        @pl.loop(0, C)
        def _(i):
            for k in range(DIM // 16):
                obuf[0, i, pl.ds(16 * k, 16)] = jnp.zeros((16,), jnp.float32)

        row0 = sid * ROWS_PT
        for k in range(ROWS_PT // C):
            pltpu.sync_copy(obuf.at[0], agg_sh.at[pl.ds(row0 + k * C, C)])
        plsc.subcore_barrier()

        @pl.loop(0, NPH)
        def _(ph):
            pltpu.sync_copy(src_hbm.at[pl.ds(base + ph * PC, PC)], src_v)
            pltpu.sync_copy(dst_hbm.at[pl.ds(base + ph * PC, PC)], dst_v)
            pltpu.sync_copy(w_hbm.at[pl.ds(base + ph * PC, PC)], w_v)

            for b in range(LOOK):
                gather(b, b)

            @pl.loop(0, PC, step=NBUF)
            def _(j):
                for k in range(NBUF):
                    jj = j + k
                    bn = (k + LOOK) % NBUF
                    ob = k % NOBUF

                    @pl.when(jj >= NOBUF)
                    def _():
                        scatter_wait(ob)

                    @pl.when(jj + LOOK < PC)
                    def _():
                        gather(jj + LOOK, bn)

                    gather_wait(k)
                    for e16 in range(C // 16):
                        wvec = w_v[jj, pl.ds(16 * e16, 16)]
                        for i in range(16):
                            e = 16 * e16 + i
                            ws = bcast_lane(wvec, i)
                            for g in range(DIM // 32):
                                pk32 = rows_v[k, e, pl.ds(16 * g, 16)]
                                pk = plsc.bitcast(pk32, jnp.bfloat16)
                                ev, od = plsc.unpack(
                                    pk, format=plsc.PackFormat.INTERLEAVED)
                                obuf[ob, e, pl.ds(32 * g, 16)] = ev * ws
                                obuf[ob, e, pl.ds(32 * g + 16, 16)] = od * ws
                    scatter(jj, ob)

            for ob in range(NOBUF):
                scatter_wait(ob)

        plsc.subcore_barrier()

        for k in range(ROWS_PT // C):
            r0 = row0 + k * C
            pltpu.sync_copy(agg_sh.at[pl.ds(r0, C)], obuf.at[0])
            pltpu.sync_copy(obuf.at[0], out_hbm.at[cid, pl.ds(r0, C)])

    return body(hb, src2d, dst2d, w2d)


BLK = 2000


def _combine_body(relu, parts_ref, h_ref, wrelp_ref, b_ref, wroot_ref,
                  o_ref, ob_ref):
    agg = parts_ref[0] + parts_ref[1]
    acc = jnp.dot(agg, wrelp_ref[...], preferred_element_type=jnp.float32,
                  precision=lax.Precision.HIGHEST)
    acc += jnp.dot(h_ref[...], wroot_ref[...], preferred_element_type=jnp.float32,
                   precision=lax.Precision.HIGHEST)
    acc += b_ref[...]
    out = jnp.maximum(acc, 0.0) if relu else acc
    o_ref[...] = out
    ob_ref[...] = out.astype(jnp.bfloat16)


def _tc_combine(parts, h, w_rel_perm, b, w_root, relu):
    return pl.pallas_call(
        functools.partial(_combine_body, relu),
        grid=(N_NODES // BLK,),
        in_specs=[
            pl.BlockSpec((NC, BLK, DIM), lambda i: (0, i, 0)),
            pl.BlockSpec((BLK, DIM), lambda i: (i, 0)),
            pl.BlockSpec((DIM, DIM), lambda i: (0, 0)),
            pl.BlockSpec((1, DIM), lambda i: (0, 0)),
            pl.BlockSpec((DIM, DIM), lambda i: (0, 0)),
        ],
        out_specs=[
            pl.BlockSpec((BLK, DIM), lambda i: (i, 0)),
            pl.BlockSpec((BLK, DIM), lambda i: (i, 0)),
        ],
        out_shape=[
            jax.ShapeDtypeStruct((N_NODES, DIM), jnp.float32),
            jax.ShapeDtypeStruct((N_NODES, DIM), jnp.bfloat16),
        ],
    )(parts, h, w_rel_perm, b, w_root)


def kernel(x, edge_index, edge_weight, W1_rel, b1, W1_root,
           W2_rel, b2, W2_root, W3_rel, b3, W3_root):
    pad = E_PAD - N_EDGES
    fill = (jnp.arange(pad, dtype=jnp.int32) * 13) % N_NODES
    src2d = jnp.concatenate([edge_index[0], fill]).reshape(CHUNKS, C)
    dst2d = jnp.concatenate([edge_index[1], fill]).reshape(CHUNKS, C)
    w2d = jnp.concatenate(
        [edge_weight, jnp.zeros((pad,), jnp.float32)]).reshape(CHUNKS, C)

    def to_pairs(hbf):
        u = lax.bitcast_convert_type(hbf, jnp.uint16).astype(jnp.uint32)
        word = u[:, : DIM // 2] | (u[:, DIM // 2:] << 16)
        return lax.bitcast_convert_type(word, jnp.int32)

    perm = jnp.asarray(_PERM)
    h = x
    hb = to_pairs(x.astype(jnp.bfloat16))
    for w_rel, b, w_root, relu in (
        (W1_rel, b1, W1_root, True),
        (W2_rel, b2, W2_root, True),
        (W3_rel, b3, W3_root, False),
    ):
        parts = _sc_aggregate(hb, src2d, dst2d, w2d)
        h, hbf = _tc_combine(parts, h, w_rel[perm, :], b.reshape(1, DIM),
                             w_root, relu)
        hb = to_pairs(hbf)
    return h


# speedup vs baseline: 1.1084x; 1.1084x over previous
import dataclasses
import functools

import numpy as np

import jax
import jax.numpy as jnp
from jax import lax
from jax.experimental import pallas as pl
from jax.experimental.pallas import tpu as pltpu
from jax.experimental.pallas import tpu_sc as plsc

N_NODES = 10000
N_EDGES = 320000
DIM = 128

NC = 2
NS = 16
NW = NC * NS

C = 64
E_PAD = 327680
CHUNKS = E_PAD // C
CPT = CHUNKS // NW
NPH = 2
PC = CPT // NPH
NBUF = 4
NOBUF = 2
LOOK = 3
AGG_ROWS = 10240
ROWS_PT = AGG_ROWS // NS

_PERM = np.array([p * 64 + g * 16 + i
                  for g in range(DIM // 32)
                  for p in range(2)
                  for i in range(16)], dtype=np.int32)


def _sc_aggregate(hb, src2d, dst2d, w2d):
    mesh = plsc.VectorSubcoreMesh(core_axis_name="c", subcore_axis_name="s")

    def bcast_lane(vec, i):
        dnums = lax.GatherDimensionNumbers(
            offset_dims=(), collapsed_slice_dims=(0,), start_index_map=(0,))
        idx = jnp.full((16, 1), i, jnp.int32)
        return lax.gather(vec, idx, dnums, (1,),
                          mode=lax.GatherScatterMode.PROMISE_IN_BOUNDS)

    cp = pltpu.CompilerParams()
    if "needs_layout_passes" in pltpu.CompilerParams.__dataclass_fields__:
        cp = dataclasses.replace(cp, needs_layout_passes=False)
    cp = dataclasses.replace(cp, use_tc_tiling_on_sc=False)

    @functools.partial(
        pl.kernel,
        out_type=jax.ShapeDtypeStruct((NC, AGG_ROWS, DIM), jnp.float32),
        mesh=mesh,
        compiler_params=cp,
        scratch_types=[
            pltpu.VMEM((PC, C), jnp.int32),
            pltpu.VMEM((PC, C), jnp.int32),
            pltpu.VMEM((PC, C), jnp.float32),
            pltpu.VMEM((NBUF, C, DIM // 2), jnp.int32),
            pltpu.VMEM((NOBUF, C, DIM), jnp.float32),
            pltpu.VMEM_SHARED((AGG_ROWS, DIM), jnp.float32),
            pltpu.SemaphoreType.DMA((NBUF,)),
            pltpu.SemaphoreType.DMA((NOBUF,)),
        ],
    )
    def body(h_hbm, src_hbm, dst_hbm, w_hbm, out_hbm,
             src_v, dst_v, w_v, rows_v, obuf, agg_sh, gsem, ssem):
        cid = lax.axis_index("c")
        sid = lax.axis_index("s")
        wid = sid * NC + cid
        base = wid * CPT

        def gather(jj, b):
            pltpu.async_copy(h_hbm.at[src_v.at[jj]], rows_v.at[b], gsem.at[b])

        def gather_wait(b):
            pltpu.make_async_copy(
                h_hbm.at[src_v.at[0]], rows_v.at[b], gsem.at[b]).wait()

        def scatter(jj, ob):
            pltpu.async_copy(obuf.at[ob], agg_sh.at[dst_v.at[jj]],
                             ssem.at[ob], add=True)

        def scatter_wait(ob):
            pltpu.make_async_copy(
                obuf.at[ob], agg_sh.at[dst_v.at[0]], ssem.at[ob]).wait()

        @pl.loop(0, C)
        def _(i):
            for k in range(DIM // 16):
                obuf[0, i, pl.ds(16 * k, 16)] = jnp.zeros((16,), jnp.float32)

        row0 = sid * ROWS_PT
        for k in range(ROWS_PT // C):
            pltpu.sync_copy(obuf.at[0], agg_sh.at[pl.ds(row0 + k * C, C)])
        plsc.subcore_barrier()

        @pl.loop(0, NPH)
        def _(ph):
            pltpu.sync_copy(src_hbm.at[pl.ds(base + ph * PC, PC)], src_v)
            pltpu.sync_copy(dst_hbm.at[pl.ds(base + ph * PC, PC)], dst_v)
            pltpu.sync_copy(w_hbm.at[pl.ds(base + ph * PC, PC)], w_v)

            for b in range(LOOK):
                gather(b, b)

            @pl.loop(0, PC, step=NBUF)
            def _(j):
                for k in range(NBUF):
                    jj = j + k
                    bn = (k + LOOK) % NBUF
                    ob = k % NOBUF

                    @pl.when(jj >= NOBUF)
                    def _():
                        scatter_wait(ob)

                    @pl.when(jj + LOOK < PC)
                    def _():
                        gather(jj + LOOK, bn)

                    gather_wait(k)
                    for e16 in range(C // 16):
                        wvec = w_v[jj, pl.ds(16 * e16, 16)]
                        for i in range(16):
                            e = 16 * e16 + i
                            ws = bcast_lane(wvec, i)
                            for g in range(DIM // 32):
                                pk32 = rows_v[k, e, pl.ds(16 * g, 16)]
                                pk = plsc.bitcast(pk32, jnp.bfloat16)
                                ev, od = plsc.unpack(
                                    pk, format=plsc.PackFormat.INTERLEAVED)
                                obuf[ob, e, pl.ds(32 * g, 16)] = ev * ws
                                obuf[ob, e, pl.ds(32 * g + 16, 16)] = od * ws
                    scatter(jj, ob)

            for ob in range(NOBUF):
                scatter_wait(ob)

        plsc.subcore_barrier()

        for k in range(ROWS_PT // C):
            r0 = row0 + k * C
            pltpu.sync_copy(agg_sh.at[pl.ds(r0, C)], obuf.at[0])
            pltpu.sync_copy(obuf.at[0], out_hbm.at[cid, pl.ds(r0, C)])

    return body(hb, src2d, dst2d, w2d)


BLK = 2000


def _combine_body(relu, parts_ref, h_ref, wrelp_ref, b_ref, wroot_ref,
                  o_ref, ob_ref):
    agg = parts_ref[0] + parts_ref[1]
    acc = jnp.dot(agg, wrelp_ref[...], preferred_element_type=jnp.float32,
                  precision=lax.Precision.HIGHEST)
    acc += jnp.dot(h_ref[...], wroot_ref[...], preferred_element_type=jnp.float32,
                   precision=lax.Precision.HIGHEST)
    acc += b_ref[...]
    out = jnp.maximum(acc, 0.0) if relu else acc
    o_ref[...] = out
    ob_ref[...] = out.astype(jnp.bfloat16)


def _tc_combine(parts, h, w_rel_perm, b, w_root, relu):
    return pl.pallas_call(
        functools.partial(_combine_body, relu),
        grid=(N_NODES // BLK,),
        in_specs=[
            pl.BlockSpec((NC, BLK, DIM), lambda i: (0, i, 0)),
            pl.BlockSpec((BLK, DIM), lambda i: (i, 0)),
            pl.BlockSpec((DIM, DIM), lambda i: (0, 0)),
            pl.BlockSpec((1, DIM), lambda i: (0, 0)),
            pl.BlockSpec((DIM, DIM), lambda i: (0, 0)),
        ],
        out_specs=[
            pl.BlockSpec((BLK, DIM), lambda i: (i, 0)),
            pl.BlockSpec((BLK, DIM), lambda i: (i, 0)),
        ],
        out_shape=[
            jax.ShapeDtypeStruct((N_NODES, DIM), jnp.float32),
            jax.ShapeDtypeStruct((N_NODES, DIM), jnp.bfloat16),
        ],
    )(parts, h, w_rel_perm, b, w_root)


def kernel(x, edge_index, edge_weight, W1_rel, b1, W1_root,
           W2_rel, b2, W2_root, W3_rel, b3, W3_root):
    pad = E_PAD - N_EDGES
    fill = (jnp.arange(pad, dtype=jnp.int32) * 13) % N_NODES
    src2d = jnp.concatenate([edge_index[0], fill]).reshape(CHUNKS, C)
    dst2d = jnp.concatenate([edge_index[1], fill]).reshape(CHUNKS, C)
    w2d = jnp.concatenate(
        [edge_weight, jnp.zeros((pad,), jnp.float32)]).reshape(CHUNKS, C)

    def to_pairs(hbf):
        u = lax.bitcast_convert_type(hbf, jnp.uint16).astype(jnp.uint32)
        word = u[:, : DIM // 2] | (u[:, DIM // 2:] << 16)
        return lax.bitcast_convert_type(word, jnp.int32)

    perm = jnp.asarray(_PERM)
    h = x
    hb = to_pairs(x.astype(jnp.bfloat16))
    for w_rel, b, w_root, relu in (
        (W1_rel, b1, W1_root, True),
        (W2_rel, b2, W2_root, True),
        (W3_rel, b3, W3_root, False),
    ):
        parts = _sc_aggregate(hb, src2d, dst2d, w2d)
        h, hbf = _tc_combine(parts, h, w_rel[perm, :], b.reshape(1, DIM),
                             w_root, relu)
        hb = to_pairs(hbf)
    return h
